# initial kernel scaffold (unmeasured)
import jax
import jax.numpy as jnp
from jax import lax
from jax.experimental import pallas as pl
from jax.experimental.pallas import tpu as pltpu

NB = 16
S_HALF = 1024


def kernel(O, Wo):
    B, S, Hs, D = O.shape
    K = Hs * D
    N = Wo.shape[1]
    BN = N // NB
    x2 = O.reshape(S, K)

    def body(x_ref, wo_ref, out_ref, recv_ref, wo_buf, send_buf, add_buf,
             wo_sems, send_sems, recv_sems, add_sems):
        my_x = lax.axis_index("x")
        my_y = lax.axis_index("y")
        my_z = lax.axis_index("z")
        partner = (1 - my_x, my_y, my_z)

        barrier = pltpu.get_barrier_semaphore()
        pl.semaphore_signal(barrier, inc=1, device_id=partner,
                            device_id_type=pl.DeviceIdType.MESH)
        pl.semaphore_wait(barrier, 1)

        own0 = my_x * S_HALF
        oth0 = (1 - my_x) * S_HALF

        def wo_dma(j, slot):
            return pltpu.make_async_copy(
                wo_ref.at[:, pl.ds(j * BN, BN)], wo_buf.at[slot],
                wo_sems.at[slot])

        wo_dma(0, 0).start()
        rdmas = []
        for j in range(NB):
            slot = j % 2
            if j + 1 < NB:
                wo_dma(j + 1, (j + 1) % 2).start()
            wo_dma(j, slot).wait()

            own = jnp.dot(x_ref[pl.ds(own0, S_HALF), :], wo_buf[slot],
                          preferred_element_type=jnp.float32)
            out_ref[:, pl.ds(j * BN, BN)] = own

            if j >= 2:
                rdmas[j - 2].wait_send()
            oth = jnp.dot(x_ref[pl.ds(oth0, S_HALF), :], wo_buf[slot],
                          preferred_element_type=jnp.float32)
            send_buf[slot] = oth

            rdma = pltpu.make_async_remote_copy(
                src_ref=send_buf.at[slot],
                dst_ref=recv_ref.at[:, pl.ds(j * BN, BN)],
                send_sem=send_sems.at[slot],
                recv_sem=recv_sems.at[j],
                device_id=partner,
                device_id_type=pl.DeviceIdType.MESH,
            )
            rdma.start()
            rdmas.append(rdma)

        rdmas[NB - 2].wait_send()
        rdmas[NB - 1].wait_send()

        def add_dma(j, slot):
            return pltpu.make_async_copy(
                recv_ref.at[:, pl.ds(j * BN, BN)], add_buf.at[slot],
                add_sems.at[slot])

        rdmas[0].wait_recv()
        add_dma(0, 0).start()
        for j in range(NB):
            slot = j % 2
            if j + 1 < NB:
                rdmas[j + 1].wait_recv()
                add_dma(j + 1, (j + 1) % 2).start()
            add_dma(j, slot).wait()
            blk = pl.ds(j * BN, BN)
            out_ref[:, blk] = out_ref[:, blk] + add_buf[slot]

    out = pl.pallas_call(
        body,
        out_shape=jax.ShapeDtypeStruct((S_HALF, N), jnp.float32),
        in_specs=[
            pl.BlockSpec(memory_space=pltpu.VMEM),
            pl.BlockSpec(memory_space=pltpu.HBM),
        ],
        out_specs=pl.BlockSpec(memory_space=pltpu.VMEM),
        scratch_shapes=[
            pltpu.HBM((S_HALF, N), jnp.float32),
            pltpu.VMEM((2, K, BN), jnp.float32),
            pltpu.VMEM((2, S_HALF, BN), jnp.float32),
            pltpu.VMEM((2, S_HALF, BN), jnp.float32),
            pltpu.SemaphoreType.DMA((2,)),
            pltpu.SemaphoreType.DMA((2,)),
            pltpu.SemaphoreType.DMA((NB,)),
            pltpu.SemaphoreType.DMA((2,)),
        ],
        compiler_params=pltpu.CompilerParams(collective_id=0),
    )(x2, Wo)
    return out.reshape(B, S_HALF, N)


# baseline (device time: 446464 ns/iter reference)
import jax
import jax.numpy as jnp
from jax import lax
from jax.experimental import pallas as pl
from jax.experimental.pallas import tpu as pltpu

jax.config.update("jax_compilation_cache_dir", "/tmp/jax_cache")
jax.config.update("jax_persistent_cache_min_compile_time_secs", 0)
jax.config.update("jax_persistent_cache_min_entry_size_bytes", 0)

NB = 32
S_HALF = 1024
LAG = 4
RING = LAG + 1


def kernel(O, Wo):
    B, S, Hs, D = O.shape
    K = Hs * D
    N = Wo.shape[1]
    BN = N // NB
    x2 = O.reshape(S, K)

    def body(x_ref, wo_ref, out_ref, recv_ref, wo_buf, own_ring, send_buf,
             add_buf, res_buf, wo_sems, send_sems, recv_sems, add_sems,
             res_sems):
        my_x = lax.axis_index("x")
        my_y = lax.axis_index("y")
        my_z = lax.axis_index("z")
        partner = (1 - my_x, my_y, my_z)

        barrier = pltpu.get_barrier_semaphore()
        pl.semaphore_signal(barrier, inc=1, device_id=partner,
                            device_id_type=pl.DeviceIdType.MESH)
        pl.semaphore_wait(barrier, 1)

        own0 = my_x * S_HALF
        oth0 = (1 - my_x) * S_HALF

        def wo_dma(j):
            return pltpu.make_async_copy(
                wo_ref.at[:, pl.ds(j * BN, BN)], wo_buf.at[j % 2],
                wo_sems.at[j % 2])

        def chunk_rdma(j):
            return pltpu.make_async_remote_copy(
                src_ref=send_buf.at[j % 2],
                dst_ref=recv_ref.at[j],
                send_sem=send_sems.at[j % 2],
                recv_sem=recv_sems.at[j],
                device_id=partner,
                device_id_type=pl.DeviceIdType.MESH,
            )

        def res_dma(i):
            return pltpu.make_async_copy(
                res_buf.at[i % 2], out_ref.at[:, pl.ds(i * BN, BN)],
                res_sems.at[i % 2])

        def finish_chunk(i):
            chunk_rdma(i).wait_recv()
            a = pltpu.make_async_copy(
                recv_ref.at[i], add_buf.at[i % 2], add_sems.at[i % 2])
            a.start()
            a.wait()

            @pl.when(i >= 2)
            def _():
                res_dma(i - 2).wait()

            res_buf[i % 2] = own_ring[i % RING] + add_buf[i % 2]
            res_dma(i).start()

        def step(j, carry):
            @pl.when(j + 1 < NB)
            def _():
                wo_dma(j + 1).start()

            wo_dma(j).wait()

            own_ring[j % RING] = jnp.dot(
                x_ref[pl.ds(own0, S_HALF), :], wo_buf[j % 2],
                preferred_element_type=jnp.float32)

            @pl.when(j >= 2)
            def _():
                chunk_rdma(j - 2).wait_send()

            send_buf[j % 2] = jnp.dot(
                x_ref[pl.ds(oth0, S_HALF), :], wo_buf[j % 2],
                preferred_element_type=jnp.float32)

            chunk_rdma(j).start()

            @pl.when(j >= LAG)
            def _():
                finish_chunk(j - LAG)

            return carry

        wo_dma(0).start()
        lax.fori_loop(0, NB, step, 0)

        chunk_rdma(NB - 2).wait_send()
        chunk_rdma(NB - 1).wait_send()

        def tail(i, carry):
            finish_chunk(i)
            return carry

        lax.fori_loop(NB - LAG, NB, tail, 0)
        res_dma(NB - 2).wait()
        res_dma(NB - 1).wait()

    out, _ = pl.pallas_call(
        body,
        out_shape=[
            jax.ShapeDtypeStruct((S_HALF, N), jnp.float32),
            jax.ShapeDtypeStruct((NB, S_HALF, BN), jnp.float32),
        ],
        in_specs=[
            pl.BlockSpec(memory_space=pltpu.VMEM),
            pl.BlockSpec(memory_space=pltpu.HBM),
        ],
        out_specs=[
            pl.BlockSpec(memory_space=pltpu.HBM),
            pl.BlockSpec(memory_space=pltpu.HBM),
        ],
        scratch_shapes=[
            pltpu.VMEM((2, K, BN), jnp.float32),
            pltpu.VMEM((RING, S_HALF, BN), jnp.float32),
            pltpu.VMEM((2, S_HALF, BN), jnp.float32),
            pltpu.VMEM((2, S_HALF, BN), jnp.float32),
            pltpu.VMEM((2, S_HALF, BN), jnp.float32),
            pltpu.SemaphoreType.DMA((2,)),
            pltpu.SemaphoreType.DMA((2,)),
            pltpu.SemaphoreType.DMA((NB,)),
            pltpu.SemaphoreType.DMA((2,)),
            pltpu.SemaphoreType.DMA((2,)),
        ],
        compiler_params=pltpu.CompilerParams(
            collective_id=0, vmem_limit_bytes=63 * 1024 * 1024),
    )(x2, Wo)
    return out.reshape(B, S_HALF, N)


# device time: 413356 ns/iter; 1.0801x vs baseline; 1.0801x over previous
import jax
import jax.numpy as jnp
from jax import lax
from jax.experimental import pallas as pl
from jax.experimental.pallas import tpu as pltpu

jax.config.update("jax_compilation_cache_dir", "/tmp/jax_cache")
jax.config.update("jax_persistent_cache_min_compile_time_secs", 0)
jax.config.update("jax_persistent_cache_min_entry_size_bytes", 0)

NB = 32
S_HALF = 1024
LAG = 4
RING = LAG + 1


def kernel(O, Wo):
    B, S, Hs, D = O.shape
    K = Hs * D
    N = Wo.shape[1]
    BN = N // NB
    o3 = O.reshape(S, Hs, D)

    def body(o_ref, wo_ref, out_ref, recv_ref, x_buf, wo_buf, own_ring,
             send_buf, add_buf, res_buf, imp_sems, wo_sems, send_sems,
             recv_sems, add_sems, res_sems):
        my_x = lax.axis_index("x")
        my_y = lax.axis_index("y")
        my_z = lax.axis_index("z")
        partner = (1 - my_x, my_y, my_z)

        imports = [
            pltpu.make_async_copy(
                o_ref.at[:, h, :], x_buf.at[:, pl.ds(h * D, D)],
                imp_sems.at[h])
            for h in range(Hs)
        ]
        for imp in imports:
            imp.start()

        barrier = pltpu.get_barrier_semaphore()
        pl.semaphore_signal(barrier, inc=1, device_id=partner,
                            device_id_type=pl.DeviceIdType.MESH)
        pl.semaphore_wait(barrier, 1)

        own0 = my_x * S_HALF
        oth0 = (1 - my_x) * S_HALF

        def wo_dma(j):
            return pltpu.make_async_copy(
                wo_ref.at[:, pl.ds(j * BN, BN)], wo_buf.at[j % 2],
                wo_sems.at[j % 2])

        def chunk_rdma(j):
            return pltpu.make_async_remote_copy(
                src_ref=send_buf.at[j % 2],
                dst_ref=recv_ref.at[j],
                send_sem=send_sems.at[j % 2],
                recv_sem=recv_sems.at[j],
                device_id=partner,
                device_id_type=pl.DeviceIdType.MESH,
            )

        def add_dma(i):
            return pltpu.make_async_copy(
                recv_ref.at[i], add_buf.at[i % 2], add_sems.at[i % 2])

        def res_dma(i):
            return pltpu.make_async_copy(
                res_buf.at[i % 2], out_ref.at[:, pl.ds(i * BN, BN)],
                res_sems.at[i % 2])

        def start_chunk(i):
            chunk_rdma(i).wait_recv()
            add_dma(i).start()

        def finish_chunk(i):
            add_dma(i).wait()

            @pl.when(i >= 2)
            def _():
                res_dma(i - 2).wait()

            res_buf[i % 2] = own_ring[i % RING] + add_buf[i % 2]
            res_dma(i).start()

        wo_dma(0).start()
        for imp in imports:
            imp.wait()

        def step(j, carry):
            @pl.when(j + 1 < NB)
            def _():
                wo_dma(j + 1).start()

            @pl.when(j >= LAG)
            def _():
                start_chunk(j - LAG)

            wo_dma(j).wait()

            own_ring[j % RING] = jnp.dot(
                x_buf[pl.ds(own0, S_HALF), :], wo_buf[j % 2],
                preferred_element_type=jnp.float32)

            @pl.when(j >= 2)
            def _():
                chunk_rdma(j - 2).wait_send()

            send_buf[j % 2] = jnp.dot(
                x_buf[pl.ds(oth0, S_HALF), :], wo_buf[j % 2],
                preferred_element_type=jnp.float32)

            chunk_rdma(j).start()

            @pl.when(j >= LAG)
            def _():
                finish_chunk(j - LAG)

            return carry

        lax.fori_loop(0, NB, step, 0)

        chunk_rdma(NB - 2).wait_send()
        chunk_rdma(NB - 1).wait_send()

        def tail(i, carry):
            start_chunk(i)
            finish_chunk(i)
            return carry

        lax.fori_loop(NB - LAG, NB, tail, 0)
        res_dma(NB - 2).wait()
        res_dma(NB - 1).wait()

    out, _ = pl.pallas_call(
        body,
        out_shape=[
            jax.ShapeDtypeStruct((S_HALF, N), jnp.float32),
            jax.ShapeDtypeStruct((NB, S_HALF, BN), jnp.float32),
        ],
        in_specs=[
            pl.BlockSpec(memory_space=pltpu.HBM),
            pl.BlockSpec(memory_space=pltpu.HBM),
        ],
        out_specs=[
            pl.BlockSpec(memory_space=pltpu.HBM),
            pl.BlockSpec(memory_space=pltpu.HBM),
        ],
        scratch_shapes=[
            pltpu.VMEM((S, K), jnp.float32),
            pltpu.VMEM((2, K, BN), jnp.float32),
            pltpu.VMEM((RING, S_HALF, BN), jnp.float32),
            pltpu.VMEM((2, S_HALF, BN), jnp.float32),
            pltpu.VMEM((2, S_HALF, BN), jnp.float32),
            pltpu.VMEM((2, S_HALF, BN), jnp.float32),
            pltpu.SemaphoreType.DMA((Hs,)),
            pltpu.SemaphoreType.DMA((2,)),
            pltpu.SemaphoreType.DMA((2,)),
            pltpu.SemaphoreType.DMA((NB,)),
            pltpu.SemaphoreType.DMA((2,)),
            pltpu.SemaphoreType.DMA((2,)),
        ],
        compiler_params=pltpu.CompilerParams(
            collective_id=0, vmem_limit_bytes=63 * 1024 * 1024),
    )(o3, Wo)
    return out.reshape(B, S_HALF, N)
